# baseline (speedup 1.0000x reference)
import functools

import jax
import jax.numpy as jnp
from jax import lax
from jax.experimental import pallas as pl
from jax.experimental.pallas import tpu as pltpu
from jax.experimental.pallas import tpu_sc as plsc

NC = 2
NS = 16
NW = NC * NS
BLK = 1024


@functools.partial(jax.jit, static_argnames=("seq", "batch", "emb"))
def _emb_lookup(x, W, *, seq, batch, emb):
    bpr = batch // BLK
    n_blocks = seq * bpr
    nb = n_blocks // NW
    mesh = plsc.VectorSubcoreMesh(
        core_axis_name="c", subcore_axis_name="s", num_cores=NC, num_subcores=NS
    )

    @functools.partial(
        pl.kernel,
        out_type=jax.ShapeDtypeStruct((seq, batch, emb), jnp.float32),
        mesh=mesh,
        scratch_types=[
            pltpu.VMEM((BLK,), jnp.int32),
            pltpu.VMEM((BLK,), jnp.int32),
            pltpu.VMEM((BLK, emb), jnp.float32),
            pltpu.VMEM((BLK, emb), jnp.float32),
            pltpu.SemaphoreType.DMA,
            pltpu.SemaphoreType.DMA,
            pltpu.SemaphoreType.DMA,
            pltpu.SemaphoreType.DMA,
            pltpu.SemaphoreType.DMA,
            pltpu.SemaphoreType.DMA,
        ],
        compiler_params=pltpu.CompilerParams(use_tc_tiling_on_sc=False),
    )
    def k(x_hbm, w_hbm, out_hbm, idx0, idx1, rows0, rows1,
          isem0, isem1, gsem0, gsem1, osem0, osem1):
        wid = lax.axis_index("s") * NC + lax.axis_index("c")
        blk0 = wid * nb

        def loc(c):
            blk = blk0 + c
            return blk // bpr, (blk % bpr) * BLK

        def fire_idx(c, idx, isem):
            s, b0 = loc(c)
            pltpu.async_copy(x_hbm.at[s, pl.ds(b0, BLK)], idx, isem)

        def wait_idx(idx, isem):
            pltpu.make_async_copy(x_hbm.at[0, pl.ds(0, BLK)], idx, isem).wait()

        def fire_gather(idx, rows, gsem):
            pltpu.async_copy(w_hbm.at[idx], rows, gsem)

        def wait_gather(idx, rows, gsem):
            pltpu.make_async_copy(w_hbm.at[idx], rows, gsem).wait()

        def fire_out(c, rows, osem):
            s, b0 = loc(c)
            pltpu.async_copy(rows, out_hbm.at[s, pl.ds(b0, BLK)], osem)

        def wait_out(rows, osem):
            pltpu.make_async_copy(rows, out_hbm.at[0, pl.ds(0, BLK)], osem).wait()

        fire_idx(0, idx0, isem0)
        wait_idx(idx0, isem0)
        fire_gather(idx0, rows0, gsem0)
        fire_idx(1, idx1, isem1)
        wait_idx(idx1, isem1)
        fire_gather(idx1, rows1, gsem1)
        wait_gather(idx0, rows0, gsem0)
        fire_out(0, rows0, osem0)
        fire_idx(2, idx0, isem0)

        def pair(k_, carry):
            c0 = 2 * k_
            wait_out(rows0, osem0)
            wait_idx(idx0, isem0)
            fire_gather(idx0, rows0, gsem0)
            wait_gather(idx1, rows1, gsem1)
            fire_out(c0 - 1, rows1, osem1)
            fire_idx(c0 + 1, idx1, isem1)
            wait_out(rows1, osem1)
            wait_idx(idx1, isem1)
            fire_gather(idx1, rows1, gsem1)
            wait_gather(idx0, rows0, gsem0)
            fire_out(c0, rows0, osem0)
            fire_idx(c0 + 2, idx0, isem0)
            return carry

        lax.fori_loop(1, (nb - 1) // 2, pair, 0)

        wait_out(rows0, osem0)
        wait_idx(idx0, isem0)
        fire_gather(idx0, rows0, gsem0)
        wait_gather(idx1, rows1, gsem1)
        fire_out(nb - 2, rows1, osem1)
        wait_gather(idx0, rows0, gsem0)
        wait_out(rows1, osem1)
        fire_out(nb - 1, rows0, osem0)
        wait_out(rows0, osem0)

    return k(x, W)


def kernel(x, W):
    x2 = x if x.ndim > 1 else x.reshape(x.shape[0], 1)
    seq, batch = x2.shape
    emb = W.shape[1]
    out = _emb_lookup(x2.astype(jnp.int32), W, seq=seq, batch=batch, emb=emb)
    return out


# speedup vs baseline: 1.0010x; 1.0010x over previous
import functools

import jax
import jax.numpy as jnp
from jax import lax
from jax.experimental import pallas as pl
from jax.experimental.pallas import tpu as pltpu
from jax.experimental.pallas import tpu_sc as plsc

NC = 2
NS = 16
NW = NC * NS
BLK = 512
NBUF = 4


@functools.partial(jax.jit, static_argnames=("seq", "batch", "emb"))
def _emb_lookup(x, W, *, seq, batch, emb):
    bpr = batch // BLK
    n_blocks = seq * bpr
    nb = n_blocks // NW
    mesh = plsc.VectorSubcoreMesh(
        core_axis_name="c", subcore_axis_name="s", num_cores=NC, num_subcores=NS
    )

    @functools.partial(
        pl.kernel,
        out_type=jax.ShapeDtypeStruct((seq, batch, emb), jnp.float32),
        mesh=mesh,
        scratch_types=[
            pltpu.VMEM((NBUF, BLK), jnp.int32),
            pltpu.VMEM((NBUF, BLK, emb), jnp.float32),
            [pltpu.SemaphoreType.DMA] * NBUF,
            [pltpu.SemaphoreType.DMA] * NBUF,
            [pltpu.SemaphoreType.DMA] * NBUF,
        ],
        compiler_params=pltpu.CompilerParams(use_tc_tiling_on_sc=False),
    )
    def k(x_hbm, w_hbm, out_hbm, idx_v, rows_v, isems, gsems, osems):
        wid = lax.axis_index("s") * NC + lax.axis_index("c")
        blk0 = wid * nb

        def loc(c):
            blk = blk0 + c
            return blk // bpr, (blk % bpr) * BLK

        def fire_idx(c, b):
            s, b0 = loc(c)
            pltpu.async_copy(x_hbm.at[s, pl.ds(b0, BLK)], idx_v.at[b], isems[b])

        def wait_idx(b):
            pltpu.make_async_copy(
                x_hbm.at[0, pl.ds(0, BLK)], idx_v.at[b], isems[b]
            ).wait()

        def fire_gather(b):
            pltpu.async_copy(w_hbm.at[idx_v.at[b]], rows_v.at[b], gsems[b])

        def wait_gather(b):
            pltpu.make_async_copy(
                w_hbm.at[idx_v.at[b]], rows_v.at[b], gsems[b]
            ).wait()

        def fire_out(c, b):
            s, b0 = loc(c)
            pltpu.async_copy(rows_v.at[b], out_hbm.at[s, pl.ds(b0, BLK)], osems[b])

        def wait_out(b):
            pltpu.make_async_copy(
                rows_v.at[b], out_hbm.at[0, pl.ds(0, BLK)], osems[b]
            ).wait()


        fire_idx(0, 0)
        fire_idx(1, 1)
        wait_idx(0)
        fire_gather(0)
        fire_idx(2, 2)
        wait_idx(1)
        fire_gather(1)
        fire_idx(3, 3)
        wait_idx(2)
        fire_gather(2)
        wait_gather(0)
        fire_out(0, 0)
        fire_idx(4, 0)
        wait_idx(3)
        fire_gather(3)
        wait_gather(1)
        fire_out(1, 1)
        fire_idx(5, 1)

        def body(k_, carry):
            for j in range(NBUF):
                c = NBUF * k_ + j
                b = j
                b2 = (j + 2) % NBUF
                wait_out(b)
                wait_idx(b)
                fire_gather(b)
                wait_gather(b2)
                fire_out(c - 2, b2)
                fire_idx(c + 2, b2)
            return carry

        n_bodies = (nb - 6) // NBUF
        lax.fori_loop(1, 1 + n_bodies, body, 0)

        for c in range(4 + NBUF * n_bodies, nb):
            b = c % NBUF
            b2 = (c + 2) % NBUF
            wait_out(b)
            wait_idx(b)
            fire_gather(b)
            wait_gather(b2)
            fire_out(c - 2, b2)
            if c + 2 <= nb - 1:
                fire_idx(c + 2, b2)

        wait_gather((nb - 2) % NBUF)
        fire_out(nb - 2, (nb - 2) % NBUF)
        wait_gather((nb - 1) % NBUF)
        fire_out(nb - 1, (nb - 1) % NBUF)
        for c in range(nb - 4, nb):
            wait_out(c % NBUF)

    return k(x, W)


def kernel(x, W):
    x2 = x if x.ndim > 1 else x.reshape(x.shape[0], 1)
    seq, batch = x2.shape
    emb = W.shape[1]
    out = _emb_lookup(x2.astype(jnp.int32), W, seq=seq, batch=batch, emb=emb)
    return out
